# baseline (speedup 1.0000x reference)
import functools

import jax
import jax.numpy as jnp
from jax import lax
from jax.experimental import pallas as pl
from jax.experimental.pallas import tpu as pltpu
from jax.experimental.pallas import tpu_sc as plsc

INPUT_DIM = 1024
LATENT_DIM = 64
NUM_EMBEDDINGS = 1024
BATCH = 16384

TILE = 512
NSEG = 4
SEG = BATCH // NSEG

_SC_INFO = plsc.get_sparse_core_info()
NW = _SC_INFO.num_cores * _SC_INFO.num_subcores
B_PER_W = SEG // NW
CHUNK = 32
NCH = B_PER_W // CHUNK


def _idx_body(x_ref, we_ref, be_ref, emb_ref, e2_ref, idx_ref):
    x = x_ref[...]
    z = jnp.dot(x, we_ref[...], preferred_element_type=jnp.float32) + be_ref[...]
    sim = jnp.dot(z, emb_ref[...], preferred_element_type=jnp.float32)
    d = jnp.sum(z * z, axis=1, keepdims=True) + e2_ref[...] - 2.0 * sim
    idx_ref[0] = jnp.argmin(d, axis=1)[None, :]


def _table_body(embt_ref, wd_ref, bd_ref, emb_ref, cb_ref, e2_ref):
    cb_ref[...] = (jnp.dot(embt_ref[...], wd_ref[...],
                           preferred_element_type=jnp.float32) + bd_ref[...])
    e2_ref[...] = jnp.sum(emb_ref[...] ** 2, axis=0, keepdims=True)


def _gather_segment(cb_hbm, idx_hbm, out_hbm, seg_base, idx_v, bufs, gsems, ssems):
    wid = lax.axis_index("s") * _SC_INFO.num_cores + lax.axis_index("c")
    wbase = wid * B_PER_W
    pltpu.sync_copy(idx_hbm.at[pl.ds(wbase, B_PER_W)], idx_v)

    def start_gather(c, b):
        return pltpu.async_copy(
            cb_hbm.at[idx_v.at[pl.ds(c * CHUNK, CHUNK)]], bufs[b], gsems[b])

    def start_store(c, b):
        return pltpu.async_copy(
            bufs[b], out_hbm.at[pl.ds(seg_base + wbase + c * CHUNK, CHUNK)],
            ssems[b])

    hs = [None, None]
    hg = [None, None]
    hg[0] = start_gather(0, 0)
    for c in range(NCH):
        b = c & 1
        nb = (c + 1) & 1
        if c + 1 < NCH:
            if hs[nb] is not None:
                hs[nb].wait()
            hg[nb] = start_gather(c + 1, nb)
        hg[b].wait()
        hs[b] = start_store(c, b)
    for b in (0, 1):
        if hs[b] is not None:
            hs[b].wait()


_SC_SCRATCH = [
    pltpu.VMEM((B_PER_W,), jnp.int32),
    pltpu.VMEM((CHUNK, INPUT_DIM), jnp.float32),
    pltpu.VMEM((CHUNK, INPUT_DIM), jnp.float32),
    pltpu.SemaphoreType.DMA,
    pltpu.SemaphoreType.DMA,
    pltpu.SemaphoreType.DMA,
    pltpu.SemaphoreType.DMA,
]

_MESH = plsc.VectorSubcoreMesh(core_axis_name="c", subcore_axis_name="s")


@functools.partial(
    pl.kernel, mesh=_MESH,
    out_type=jax.ShapeDtypeStruct((BATCH, INPUT_DIM), jnp.float32),
    scratch_types=_SC_SCRATCH,
)
def _sc_gather_seed(cb_hbm, idx_hbm, out_hbm, idx_v, b0, b1, g0, g1, s0, s1):
    _gather_segment(cb_hbm, idx_hbm, out_hbm, 0, idx_v,
                    (b0, b1), (g0, g1), (s0, s1))


def _make_sc_gather_inplace(seg_base):
    @functools.partial(pl.kernel, mesh=_MESH, out_type=(),
                       scratch_types=_SC_SCRATCH)
    def f(cb_hbm, idx_hbm, out_hbm, idx_v, b0, b1, g0, g1, s0, s1):
        _gather_segment(cb_hbm, idx_hbm, out_hbm, seg_base, idx_v,
                        (b0, b1), (g0, g1), (s0, s1))
    return f


_sc_gather_inplace = [_make_sc_gather_inplace(k * SEG) for k in range(1, NSEG)]


@jax.jit
def kernel(x, W_enc, b_enc, W_emb, W_dec, b_dec):
    full = lambda shape: pl.BlockSpec(shape, lambda i: (0,) * len(shape))
    nbs = SEG // TILE

    cb, e2 = pl.pallas_call(
        _table_body,
        in_specs=[
            pl.BlockSpec((NUM_EMBEDDINGS, LATENT_DIM), lambda: (0, 0)),
            pl.BlockSpec((LATENT_DIM, INPUT_DIM), lambda: (0, 0)),
            pl.BlockSpec((1, INPUT_DIM), lambda: (0, 0)),
            pl.BlockSpec((LATENT_DIM, NUM_EMBEDDINGS), lambda: (0, 0)),
        ],
        out_specs=[
            pl.BlockSpec((NUM_EMBEDDINGS, INPUT_DIM), lambda: (0, 0)),
            pl.BlockSpec((1, NUM_EMBEDDINGS), lambda: (0, 0)),
        ],
        out_shape=[
            jax.ShapeDtypeStruct((NUM_EMBEDDINGS, INPUT_DIM), jnp.float32),
            jax.ShapeDtypeStruct((1, NUM_EMBEDDINGS), jnp.float32),
        ],
    )(W_emb.T, W_dec, b_dec.reshape(1, -1), W_emb)

    def idx_call(k):
        base = k * nbs
        return pl.pallas_call(
            _idx_body,
            grid=(nbs,),
            in_specs=[
                pl.BlockSpec((TILE, INPUT_DIM), lambda i: (base + i, 0)),
                full((INPUT_DIM, LATENT_DIM)),
                full((1, LATENT_DIM)),
                full((LATENT_DIM, NUM_EMBEDDINGS)),
                full((1, NUM_EMBEDDINGS)),
            ],
            out_specs=pl.BlockSpec((1, 1, TILE), lambda i: (i, 0, 0)),
            out_shape=jax.ShapeDtypeStruct((nbs, 1, TILE), jnp.int32),
        )
    be = b_enc.reshape(1, -1)

    idx = [idx_call(k)(x, W_enc, be, W_emb, e2).reshape(SEG) for k in range(NSEG)]

    out0 = _sc_gather_seed(cb, idx[0])
    oref = jax.new_ref(out0)
    for k in range(1, NSEG):
        _sc_gather_inplace[k - 1](cb, idx[k], oref)
    return oref[...]


# speedup vs baseline: 1.4144x; 1.4144x over previous
import jax
import jax.numpy as jnp
from jax import lax
from jax.experimental import pallas as pl

INPUT_DIM = 1024
LATENT_DIM = 64
NUM_EMBEDDINGS = 1024
BATCH = 16384

TILE = 1024


def _e2_body(emb_ref, e2_ref):
    e2_ref[...] = jnp.sum(emb_ref[...] ** 2, axis=0, keepdims=True)


def _vq_body(x_ref, we_ref, be_ref, emb_ref, e2_ref, wd_ref, bd_ref, out_ref):
    x = x_ref[...]
    z = jnp.dot(x, we_ref[...], preferred_element_type=jnp.float32) + be_ref[...]
    sim = jnp.dot(z, emb_ref[...], preferred_element_type=jnp.float32)
    d = jnp.sum(z * z, axis=1, keepdims=True) + e2_ref[...] - 2.0 * sim
    idx = jnp.argmin(d, axis=1)
    enc = (lax.broadcasted_iota(jnp.int32, (TILE, NUM_EMBEDDINGS), 1)
           == idx[:, None]).astype(jnp.float32)
    q = lax.dot_general(enc, emb_ref[...], (((1,), (1,)), ((), ())),
                        preferred_element_type=jnp.float32)
    out_ref[...] = (jnp.dot(q, wd_ref[...], preferred_element_type=jnp.float32)
                    + bd_ref[...])


@jax.jit
def kernel(x, W_enc, b_enc, W_emb, W_dec, b_dec):
    nb = BATCH // TILE
    full = lambda shape: pl.BlockSpec(shape, lambda i: (0,) * len(shape))
    e2 = pl.pallas_call(
        _e2_body,
        in_specs=[pl.BlockSpec((LATENT_DIM, NUM_EMBEDDINGS), lambda: (0, 0))],
        out_specs=pl.BlockSpec((1, NUM_EMBEDDINGS), lambda: (0, 0)),
        out_shape=jax.ShapeDtypeStruct((1, NUM_EMBEDDINGS), jnp.float32),
    )(W_emb)
    out = pl.pallas_call(
        _vq_body,
        grid=(nb,),
        in_specs=[
            pl.BlockSpec((TILE, INPUT_DIM), lambda i: (i, 0)),
            full((INPUT_DIM, LATENT_DIM)),
            full((1, LATENT_DIM)),
            full((LATENT_DIM, NUM_EMBEDDINGS)),
            full((1, NUM_EMBEDDINGS)),
            full((LATENT_DIM, INPUT_DIM)),
            full((1, INPUT_DIM)),
        ],
        out_specs=pl.BlockSpec((TILE, INPUT_DIM), lambda i: (i, 0)),
        out_shape=jax.ShapeDtypeStruct((BATCH, INPUT_DIM), jnp.float32),
    )(x, W_enc, b_enc.reshape(1, -1), W_emb, e2, W_dec, b_dec.reshape(1, -1))
    return out
